# baseline (speedup 1.0000x reference)
import functools

import jax
import jax.numpy as jnp
from jax import lax
from jax.experimental import pallas as pl
from jax.experimental.pallas import tpu as pltpu
from jax.experimental.pallas import tpu_sc as plsc

F, B, V, D = 26, 16384, 100000, 32
NC, NS = 2, 16
NW = NC * NS
BPW = B // NW
CHUNK = 128
NCH = BPW // CHUNK


def _sc_body(x_hbm, tbl_hbm, bias_hbm, out_hbm, xbuf, biasbuf, idxc, rowc, sem):
    wid = lax.axis_index("s") * NC + lax.axis_index("c")
    b0 = wid * BPW

    pltpu.sync_copy(x_hbm.at[:, pl.ds(b0, BPW)], xbuf)
    pltpu.sync_copy(bias_hbm, biasbuf)

    def per_feature(f, _):
        fV = f * V
        bv0 = biasbuf[f, pl.ds(0, 16)]
        bv1 = biasbuf[f, pl.ds(16, 16)]

        def per_chunk(c, _):
            def build(g, _):
                v = xbuf[f, pl.ds(c * CHUNK + g * 16, 16)] + fV
                idxc[pl.ds(g * 16, 16)] = v
                return 0

            lax.fori_loop(0, CHUNK // 16, build, 0)

            pltpu.async_copy(tbl_hbm.at[idxc], rowc, sem).wait()

            def bias_add(j, _):
                rowc[j, pl.ds(0, 16)] = rowc[j, pl.ds(0, 16)] + bv0
                rowc[j, pl.ds(16, 16)] = rowc[j, pl.ds(16, 16)] + bv1
                return 0

            lax.fori_loop(0, CHUNK, bias_add, 0)

            pltpu.sync_copy(
                rowc, out_hbm.at[pl.ds(b0 + c * CHUNK, CHUNK), pl.ds(f * D, D)]
            )
            return 0

        lax.fori_loop(0, NCH, per_chunk, 0)
        return 0

    lax.fori_loop(0, F, per_feature, 0)


@functools.partial(
    pl.kernel,
    out_type=jax.ShapeDtypeStruct((B, F * D), jnp.float32),
    mesh=plsc.VectorSubcoreMesh(core_axis_name="c", subcore_axis_name="s"),
    scratch_types=[
        pltpu.VMEM((F, BPW), jnp.int32),
        pltpu.VMEM((F, D), jnp.float32),
        pltpu.VMEM((CHUNK,), jnp.int32),
        pltpu.VMEM((CHUNK, D), jnp.float32),
        pltpu.SemaphoreType.DMA,
    ],
    compiler_params=pltpu.CompilerParams(use_tc_tiling_on_sc=False),
)
def _tokenize_sc(x_hbm, tbl_hbm, bias_hbm, out_hbm, xbuf, biasbuf, idxc, rowc, sem):
    _sc_body(x_hbm, tbl_hbm, bias_hbm, out_hbm, xbuf, biasbuf, idxc, rowc, sem)


def kernel(x_dict, tables, bias):
    x32 = x_dict.astype(jnp.int32)
    tbl = tables.reshape(F * V, D)
    out = _tokenize_sc(x32, tbl, bias)
    return out.reshape(B, F, D)


# speedup vs baseline: 1.0938x; 1.0938x over previous
import functools

import jax
import jax.numpy as jnp
from jax import lax
from jax.experimental import pallas as pl
from jax.experimental.pallas import tpu as pltpu
from jax.experimental.pallas import tpu_sc as plsc

F, B, V, D = 26, 16384, 100000, 32
NC, NS = 2, 16
NW = NC * NS
BPW = B // NW
CHUNK = 128
NCH = BPW // CHUNK


def _sc_body(x_hbm, tbl_hbm, bias_hbm, out_hbm, xbuf, biasbuf, idxall, rows, gsem, osem):
    wid = lax.axis_index("s") * NC + lax.axis_index("c")
    b0 = wid * BPW

    pltpu.sync_copy(x_hbm.at[:, pl.ds(b0, BPW)], xbuf)
    pltpu.sync_copy(bias_hbm, biasbuf)

    def build_f(f, _):
        fV = f * V

        def build_g(i, _):
            idxall[f, pl.ds(i * 16, 16)] = xbuf[f, pl.ds(i * 16, 16)] + fV
            return 0

        return lax.fori_loop(0, BPW // 16, build_g, 0)

    lax.fori_loop(0, F, build_f, 0)

    def gather4(f, p):
        for c in range(NCH):
            pltpu.async_copy(
                tbl_hbm.at[idxall.at[f, pl.ds(c * CHUNK, CHUNK)]],
                rows.at[p * NCH + c],
                gsem,
            )

    def drain_gathers():
        for c in range(NCH):
            pltpu.make_async_copy(
                tbl_hbm.at[pl.ds(0, CHUNK)], rows.at[c], gsem
            ).wait()

    def drain_writes():
        for c in range(NCH):
            pltpu.make_async_copy(
                rows.at[c], out_hbm.at[pl.ds(b0, CHUNK), pl.ds(0, D)], osem
            ).wait()

    def process4(f, p):
        bv0 = biasbuf[f, pl.ds(0, 16)]
        bv1 = biasbuf[f, pl.ds(16, 16)]
        for c in range(NCH):
            s = p * NCH + c

            @plsc.parallel_loop(0, CHUNK, 1, unroll=8)
            def _bias(j):
                rows[s, j, pl.ds(0, 16)] = rows[s, j, pl.ds(0, 16)] + bv0
                rows[s, j, pl.ds(16, 16)] = rows[s, j, pl.ds(16, 16)] + bv1

            pltpu.async_copy(
                rows.at[s],
                out_hbm.at[pl.ds(b0 + c * CHUNK, CHUNK), pl.ds(f * D, D)],
                osem,
            )

    gather4(0, 0)

    def outer(f2, _):
        f0 = 2 * f2
        drain_gathers()

        @pl.when(f2 > 0)
        def _():
            drain_writes()

        gather4(f0 + 1, 1)
        process4(f0, 0)
        drain_gathers()
        drain_writes()

        @pl.when(f2 < F // 2 - 1)
        def _():
            gather4(f0 + 2, 0)

        process4(f0 + 1, 1)
        return 0

    lax.fori_loop(0, F // 2, outer, 0)
    drain_writes()


@functools.partial(
    pl.kernel,
    out_type=jax.ShapeDtypeStruct((B, F * D), jnp.float32),
    mesh=plsc.VectorSubcoreMesh(core_axis_name="c", subcore_axis_name="s"),
    scratch_types=[
        pltpu.VMEM((F, BPW), jnp.int32),
        pltpu.VMEM((F, D), jnp.float32),
        pltpu.VMEM((F, BPW), jnp.int32),
        pltpu.VMEM((2 * NCH, CHUNK, D), jnp.float32),
        pltpu.SemaphoreType.DMA,
        pltpu.SemaphoreType.DMA,
    ],
    compiler_params=pltpu.CompilerParams(use_tc_tiling_on_sc=False),
)
def _tokenize_sc(x_hbm, tbl_hbm, bias_hbm, out_hbm, xbuf, biasbuf, idxall, rows, gsem, osem):
    _sc_body(x_hbm, tbl_hbm, bias_hbm, out_hbm, xbuf, biasbuf, idxall, rows, gsem, osem)


def kernel(x_dict, tables, bias):
    x32 = x_dict.astype(jnp.int32)
    tbl = tables.reshape(F * V, D)
    out = _tokenize_sc(x32, tbl, bias)
    return out.reshape(B, F, D)
